# speedup vs baseline: 5.2169x; 5.2169x over previous
import functools

import jax
import jax.numpy as jnp
from jax import lax
from jax.experimental import pallas as pl
from jax.experimental.pallas import tpu as pltpu
from jax.experimental.pallas import tpu_sc as plsc

N_NODES = 10000
D_FEAT = 128
N_EDGES = 320000

N_CORES = 2
N_TILES = 16
N_WORKERS = N_CORES * N_TILES

N_PAD = 10240
CHUNK = 80
EDGES_PER_TILE = N_EDGES // N_WORKERS
N_CHUNKS = EDGES_PER_TILE // CHUNK
ROWS_PER_TILE = N_PAD // N_TILES


def _sc_partials(x, src, dst, zrows):
    mesh = plsc.VectorSubcoreMesh(core_axis_name="c", subcore_axis_name="s")

    @functools.partial(
        pl.kernel,
        mesh=mesh,
        out_type=jax.ShapeDtypeStruct((N_CORES, N_PAD, D_FEAT), jnp.float32),
        scratch_types=[
            pltpu.VMEM((CHUNK,), jnp.int32),
            pltpu.VMEM((CHUNK,), jnp.int32),
            pltpu.VMEM((CHUNK, D_FEAT), jnp.float32),
            pltpu.VMEM_SHARED((N_PAD, D_FEAT), jnp.float32),
            pltpu.SemaphoreType.DMA,
        ],
    )
    def k(x_hbm, src_hbm, dst_hbm, z_hbm, out_hbm, src_v, dst_v, rows_v, acc, sem):
        c = lax.axis_index("c")
        s = lax.axis_index("s")
        wid = s * N_CORES + c

        pltpu.sync_copy(z_hbm, acc.at[pl.ds(s * ROWS_PER_TILE, ROWS_PER_TILE)])
        plsc.subcore_barrier()

        ebase = wid * EDGES_PER_TILE

        def body(i, carry):
            e0 = ebase + i * CHUNK
            pltpu.sync_copy(src_hbm.at[pl.ds(e0, CHUNK)], src_v)
            pltpu.sync_copy(dst_hbm.at[pl.ds(e0, CHUNK)], dst_v)
            pltpu.async_copy(x_hbm.at[src_v], rows_v, sem).wait()
            pltpu.sync_copy(rows_v, acc.at[dst_v], add=True)
            return carry

        lax.fori_loop(0, N_CHUNKS, body, 0)

        plsc.subcore_barrier()
        pltpu.sync_copy(
            acc.at[pl.ds(s * ROWS_PER_TILE, ROWS_PER_TILE)],
            out_hbm.at[c, pl.ds(s * ROWS_PER_TILE, ROWS_PER_TILE)],
        )

    return k(x, src, dst, zrows)


def _combine(partials):
    blk = 256

    def body(p_ref, o_ref):
        o_ref[...] = p_ref[0] + p_ref[1]

    return pl.pallas_call(
        body,
        grid=(N_PAD // blk,),
        in_specs=[pl.BlockSpec((N_CORES, blk, D_FEAT), lambda i: (0, i, 0))],
        out_specs=pl.BlockSpec((blk, D_FEAT), lambda i: (i, 0)),
        out_shape=jax.ShapeDtypeStruct((N_PAD, D_FEAT), jnp.float32),
    )(partials)


def kernel(x, edge_index):
    src = edge_index[0]
    dst = edge_index[1]
    zrows = jnp.zeros((ROWS_PER_TILE, D_FEAT), jnp.float32)
    partials = _sc_partials(x, src, dst, zrows)
    out = _combine(partials)
    return out[:N_NODES]


# speedup vs baseline: 8.5148x; 1.6321x over previous
import functools

import jax
import jax.numpy as jnp
from jax import lax
from jax.experimental import pallas as pl
from jax.experimental.pallas import tpu as pltpu
from jax.experimental.pallas import tpu_sc as plsc

N_NODES = 10000
D_FEAT = 128
N_EDGES = 320000

N_CORES = 2
N_TILES = 16
N_WORKERS = N_CORES * N_TILES

N_PAD = 10240
CHUNK = 80
EDGES_PER_TILE = N_EDGES // N_WORKERS
N_CHUNKS = EDGES_PER_TILE // CHUNK
SEG = 25
N_SEGS = N_CHUNKS // SEG
SEG_PAIRS = (SEG - 1) // 2
ROWS_PER_TILE = N_PAD // N_TILES


def _sc_partials(x, src3, dst3, zrows):
    mesh = plsc.VectorSubcoreMesh(core_axis_name="c", subcore_axis_name="s")

    @functools.partial(
        pl.kernel,
        mesh=mesh,
        out_type=jax.ShapeDtypeStruct((N_CORES, N_PAD, D_FEAT), jnp.float32),
        scratch_types=[
            pltpu.VMEM((SEG, CHUNK), jnp.int32),
            pltpu.VMEM((SEG, CHUNK), jnp.int32),
            pltpu.VMEM((CHUNK, D_FEAT), jnp.float32),
            pltpu.VMEM((CHUNK, D_FEAT), jnp.float32),
            pltpu.VMEM_SHARED((N_PAD, D_FEAT), jnp.float32),
            pltpu.SemaphoreType.DMA,
            pltpu.SemaphoreType.DMA,
            pltpu.SemaphoreType.DMA,
            pltpu.SemaphoreType.DMA,
        ],
    )
    def k(
        x_hbm, src_hbm, dst_hbm, z_hbm, out_hbm,
        src_t, dst_t, rows0, rows1, acc, g0, g1, s0, s1,
    ):
        c = lax.axis_index("c")
        s = lax.axis_index("s")
        wid = s * N_CORES + c

        pltpu.sync_copy(z_hbm, acc.at[pl.ds(s * ROWS_PER_TILE, ROWS_PER_TILE)])
        plsc.subcore_barrier()

        for seg in range(N_SEGS):
            pltpu.sync_copy(src_hbm.at[wid, seg], src_t)
            pltpu.sync_copy(dst_hbm.at[wid, seg], dst_t)

            pltpu.async_copy(x_hbm.at[src_t.at[0]], rows0, g0)

            def body(i, carry):
                a = 2 * i
                b = a + 1
                pltpu.make_async_copy(x_hbm.at[src_t.at[a]], rows0, g0).wait()
                @pl.when(i > 0)
                def _():
                    pltpu.make_async_copy(
                        rows1, acc.at[dst_t.at[a - 1]], s1
                    ).wait()
                pltpu.async_copy(x_hbm.at[src_t.at[b]], rows1, g1)
                pltpu.async_copy(rows0, acc.at[dst_t.at[a]], s0, add=True)
                pltpu.make_async_copy(x_hbm.at[src_t.at[b]], rows1, g1).wait()
                pltpu.make_async_copy(rows0, acc.at[dst_t.at[a]], s0).wait()
                @pl.when(i < SEG_PAIRS - 1)
                def _():
                    pltpu.async_copy(x_hbm.at[src_t.at[a + 2]], rows0, g0)
                pltpu.async_copy(rows1, acc.at[dst_t.at[b]], s1, add=True)
                return carry

            lax.fori_loop(0, SEG_PAIRS, body, 0)

            last = SEG - 1
            pltpu.async_copy(x_hbm.at[src_t.at[last]], rows0, g0)
            pltpu.make_async_copy(x_hbm.at[src_t.at[last]], rows0, g0).wait()
            pltpu.make_async_copy(rows1, acc.at[dst_t.at[last - 1]], s1).wait()
            pltpu.sync_copy(rows0, acc.at[dst_t.at[last]], add=True)

        plsc.subcore_barrier()
        pltpu.sync_copy(
            acc.at[pl.ds(s * ROWS_PER_TILE, ROWS_PER_TILE)],
            out_hbm.at[c, pl.ds(s * ROWS_PER_TILE, ROWS_PER_TILE)],
        )

    return k(x, src3, dst3, zrows)


def _combine(partials):
    blk = 256

    def body(p_ref, o_ref):
        o_ref[...] = p_ref[0] + p_ref[1]

    return pl.pallas_call(
        body,
        grid=(N_PAD // blk,),
        in_specs=[pl.BlockSpec((N_CORES, blk, D_FEAT), lambda i: (0, i, 0))],
        out_specs=pl.BlockSpec((blk, D_FEAT), lambda i: (i, 0)),
        out_shape=jax.ShapeDtypeStruct((N_PAD, D_FEAT), jnp.float32),
    )(partials)


def kernel(x, edge_index):
    src3 = edge_index[0].reshape(N_WORKERS, N_SEGS, SEG, CHUNK)
    dst3 = edge_index[1].reshape(N_WORKERS, N_SEGS, SEG, CHUNK)
    zrows = jnp.zeros((ROWS_PER_TILE, D_FEAT), jnp.float32)
    partials = _sc_partials(x, src3, dst3, zrows)
    out = _combine(partials)
    return out[:N_NODES]


# speedup vs baseline: 9.2430x; 1.0855x over previous
import functools

import jax
import jax.numpy as jnp
from jax import lax
from jax.experimental import pallas as pl
from jax.experimental.pallas import tpu as pltpu
from jax.experimental.pallas import tpu_sc as plsc

N_NODES = 10000
D_FEAT = 128
N_EDGES = 320000

N_CORES = 2
N_TILES = 16
N_WORKERS = N_CORES * N_TILES

N_PAD = 10240
CHUNK = 100
EDGES_PER_TILE = N_EDGES // N_WORKERS
N_CHUNKS = EDGES_PER_TILE // CHUNK
SEG = 25
N_SEGS = N_CHUNKS // SEG
SEG_PAIRS = (SEG - 1) // 2
ROWS_PER_TILE = N_PAD // N_TILES


def _sc_partials(x, src3, dst3, zrows):
    mesh = plsc.VectorSubcoreMesh(core_axis_name="c", subcore_axis_name="s")

    @functools.partial(
        pl.kernel,
        mesh=mesh,
        out_type=jax.ShapeDtypeStruct((N_CORES, N_PAD, D_FEAT), jnp.float32),
        scratch_types=[
            pltpu.VMEM((SEG, CHUNK), jnp.int32),
            pltpu.VMEM((SEG, CHUNK), jnp.int32),
            pltpu.VMEM((CHUNK, D_FEAT), jnp.float32),
            pltpu.VMEM((CHUNK, D_FEAT), jnp.float32),
            pltpu.VMEM_SHARED((N_PAD, D_FEAT), jnp.float32),
            pltpu.SemaphoreType.DMA,
            pltpu.SemaphoreType.DMA,
            pltpu.SemaphoreType.DMA,
            pltpu.SemaphoreType.DMA,
        ],
    )
    def k(
        x_hbm, src_hbm, dst_hbm, z_hbm, out_hbm,
        src_t, dst_t, rows0, rows1, acc, g0, g1, s0, s1,
    ):
        c = lax.axis_index("c")
        s = lax.axis_index("s")
        wid = s * N_CORES + c

        pltpu.sync_copy(z_hbm, acc.at[pl.ds(s * ROWS_PER_TILE, ROWS_PER_TILE)])
        plsc.subcore_barrier()

        for seg in range(N_SEGS):
            pltpu.sync_copy(src_hbm.at[wid, seg], src_t)
            pltpu.sync_copy(dst_hbm.at[wid, seg], dst_t)

            pltpu.async_copy(x_hbm.at[src_t.at[0]], rows0, g0)

            def body(i, carry):
                a = 2 * i
                b = a + 1
                pltpu.make_async_copy(x_hbm.at[src_t.at[a]], rows0, g0).wait()
                @pl.when(i > 0)
                def _():
                    pltpu.make_async_copy(
                        rows1, acc.at[dst_t.at[a - 1]], s1
                    ).wait()
                pltpu.async_copy(x_hbm.at[src_t.at[b]], rows1, g1)
                pltpu.async_copy(rows0, acc.at[dst_t.at[a]], s0, add=True)
                pltpu.make_async_copy(x_hbm.at[src_t.at[b]], rows1, g1).wait()
                pltpu.make_async_copy(rows0, acc.at[dst_t.at[a]], s0).wait()
                @pl.when(i < SEG_PAIRS - 1)
                def _():
                    pltpu.async_copy(x_hbm.at[src_t.at[a + 2]], rows0, g0)
                pltpu.async_copy(rows1, acc.at[dst_t.at[b]], s1, add=True)
                return carry

            lax.fori_loop(0, SEG_PAIRS, body, 0)

            last = SEG - 1
            pltpu.async_copy(x_hbm.at[src_t.at[last]], rows0, g0)
            pltpu.make_async_copy(x_hbm.at[src_t.at[last]], rows0, g0).wait()
            pltpu.make_async_copy(rows1, acc.at[dst_t.at[last - 1]], s1).wait()
            pltpu.sync_copy(rows0, acc.at[dst_t.at[last]], add=True)

        plsc.subcore_barrier()
        pltpu.sync_copy(
            acc.at[pl.ds(s * ROWS_PER_TILE, ROWS_PER_TILE)],
            out_hbm.at[c, pl.ds(s * ROWS_PER_TILE, ROWS_PER_TILE)],
        )

    return k(x, src3, dst3, zrows)


def _combine(partials):
    blk = 256

    def body(p_ref, o_ref):
        o_ref[...] = p_ref[0] + p_ref[1]

    return pl.pallas_call(
        body,
        grid=(N_PAD // blk,),
        in_specs=[pl.BlockSpec((N_CORES, blk, D_FEAT), lambda i: (0, i, 0))],
        out_specs=pl.BlockSpec((blk, D_FEAT), lambda i: (i, 0)),
        out_shape=jax.ShapeDtypeStruct((N_PAD, D_FEAT), jnp.float32),
    )(partials)


def kernel(x, edge_index):
    src3 = edge_index[0].reshape(N_WORKERS, N_SEGS, SEG, CHUNK)
    dst3 = edge_index[1].reshape(N_WORKERS, N_SEGS, SEG, CHUNK)
    zrows = jnp.zeros((ROWS_PER_TILE, D_FEAT), jnp.float32)
    partials = _sc_partials(x, src3, dst3, zrows)
    out = _combine(partials)
    return out[:N_NODES]


# speedup vs baseline: 10.3533x; 1.1201x over previous
import functools

import jax
import jax.numpy as jnp
from jax import lax
from jax.experimental import pallas as pl
from jax.experimental.pallas import tpu as pltpu
from jax.experimental.pallas import tpu_sc as plsc

N_NODES = 10000
D_FEAT = 128
N_EDGES = 320000

N_CORES = 2
N_TILES = 16
N_WORKERS = N_CORES * N_TILES

N_PAD = 10240
CHUNK = 100
EDGES_PER_TILE = N_EDGES // N_WORKERS
N_CHUNKS = EDGES_PER_TILE // CHUNK
SEG = 25
N_SEGS = N_CHUNKS // SEG
SEG_PAIRS = (SEG - 1) // 2
ROWS_PER_TILE = N_PAD // N_TILES


def _sc_partials(x, idx5, zrows):
    mesh = plsc.VectorSubcoreMesh(core_axis_name="c", subcore_axis_name="s")

    @functools.partial(
        pl.kernel,
        mesh=mesh,
        out_type=jax.ShapeDtypeStruct((N_CORES, N_PAD, D_FEAT), jnp.float32),
        scratch_types=[
            pltpu.VMEM((SEG, CHUNK), jnp.int32),
            pltpu.VMEM((SEG, CHUNK), jnp.int32),
            pltpu.VMEM((CHUNK, D_FEAT), jnp.float32),
            pltpu.VMEM((CHUNK, D_FEAT), jnp.float32),
            pltpu.VMEM_SHARED((N_PAD, D_FEAT), jnp.float32),
            pltpu.SemaphoreType.DMA,
            pltpu.SemaphoreType.DMA,
            pltpu.SemaphoreType.DMA,
            pltpu.SemaphoreType.DMA,
        ],
    )
    def k(
        x_hbm, idx_hbm, z_hbm, out_hbm,
        src_t, dst_t, rows0, rows1, acc, g0, g1, s0, s1,
    ):
        c = lax.axis_index("c")
        s = lax.axis_index("s")
        wid = s * N_CORES + c

        pltpu.sync_copy(z_hbm, acc.at[pl.ds(s * ROWS_PER_TILE, ROWS_PER_TILE)])
        plsc.subcore_barrier()

        for seg in range(N_SEGS):
            pltpu.sync_copy(idx_hbm.at[0, wid, seg], src_t)
            pltpu.sync_copy(idx_hbm.at[1, wid, seg], dst_t)

            pltpu.async_copy(x_hbm.at[src_t.at[0]], rows0, g0)

            def body(i, carry):
                a = 2 * i
                b = a + 1
                pltpu.make_async_copy(x_hbm.at[src_t.at[a]], rows0, g0).wait()
                @pl.when(i > 0)
                def _():
                    pltpu.make_async_copy(
                        rows1, acc.at[dst_t.at[a - 1]], s1
                    ).wait()
                pltpu.async_copy(x_hbm.at[src_t.at[b]], rows1, g1)
                pltpu.async_copy(rows0, acc.at[dst_t.at[a]], s0, add=True)
                pltpu.make_async_copy(x_hbm.at[src_t.at[b]], rows1, g1).wait()
                pltpu.make_async_copy(rows0, acc.at[dst_t.at[a]], s0).wait()
                @pl.when(i < SEG_PAIRS - 1)
                def _():
                    pltpu.async_copy(x_hbm.at[src_t.at[a + 2]], rows0, g0)
                pltpu.async_copy(rows1, acc.at[dst_t.at[b]], s1, add=True)
                return carry

            lax.fori_loop(0, SEG_PAIRS, body, 0)

            last = SEG - 1
            pltpu.async_copy(x_hbm.at[src_t.at[last]], rows0, g0)
            pltpu.make_async_copy(x_hbm.at[src_t.at[last]], rows0, g0).wait()
            pltpu.make_async_copy(rows1, acc.at[dst_t.at[last - 1]], s1).wait()
            pltpu.sync_copy(rows0, acc.at[dst_t.at[last]], add=True)

        plsc.subcore_barrier()
        pltpu.sync_copy(
            acc.at[pl.ds(s * ROWS_PER_TILE, ROWS_PER_TILE)],
            out_hbm.at[c, pl.ds(s * ROWS_PER_TILE, ROWS_PER_TILE)],
        )

    return k(x, idx5, zrows)


def _combine(partials):
    blk = 400

    def body(p_ref, o_ref):
        o_ref[...] = p_ref[0] + p_ref[1]

    return pl.pallas_call(
        body,
        grid=(N_NODES // blk,),
        in_specs=[pl.BlockSpec((N_CORES, blk, D_FEAT), lambda i: (0, i, 0))],
        out_specs=pl.BlockSpec((blk, D_FEAT), lambda i: (i, 0)),
        out_shape=jax.ShapeDtypeStruct((N_NODES, D_FEAT), jnp.float32),
    )(partials)


def kernel(x, edge_index):
    idx5 = edge_index.reshape(2, N_WORKERS, N_SEGS, SEG, CHUNK)
    zrows = jnp.zeros((ROWS_PER_TILE, D_FEAT), jnp.float32)
    partials = _sc_partials(x, idx5, zrows)
    return _combine(partials)


# speedup vs baseline: 12.7208x; 1.2287x over previous
import functools

import jax
import jax.numpy as jnp
from jax import lax
from jax.experimental import pallas as pl
from jax.experimental.pallas import tpu as pltpu
from jax.experimental.pallas import tpu_sc as plsc

N_NODES = 10000
D_FEAT = 128
N_EDGES = 320000

N_CORES = 2
N_TILES = 16
N_WORKERS = N_CORES * N_TILES

N_PAD = 10240
CHUNK = 80
EDGES_PER_TILE = N_EDGES // N_WORKERS
N_CHUNKS = EDGES_PER_TILE // CHUNK
SEG = 25
N_SEGS = N_CHUNKS // SEG
RING_TRIPS = (SEG - 4) // 3
ROWS_PER_TILE = N_PAD // N_TILES


def _sc_partials(x, idx5, zrows):
    mesh = plsc.VectorSubcoreMesh(core_axis_name="c", subcore_axis_name="s")

    @functools.partial(
        pl.kernel,
        mesh=mesh,
        out_type=jax.ShapeDtypeStruct((N_CORES, N_PAD, D_FEAT), jnp.float32),
        scratch_types=[
            pltpu.VMEM((SEG, CHUNK), jnp.int32),
            pltpu.VMEM((SEG, CHUNK), jnp.int32),
            pltpu.VMEM((CHUNK, D_FEAT), jnp.float32),
            pltpu.VMEM((CHUNK, D_FEAT), jnp.float32),
            pltpu.VMEM((CHUNK, D_FEAT), jnp.float32),
            pltpu.VMEM_SHARED((N_PAD, D_FEAT), jnp.float32),
            pltpu.SemaphoreType.DMA,
            pltpu.SemaphoreType.DMA,
            pltpu.SemaphoreType.DMA,
            pltpu.SemaphoreType.DMA,
            pltpu.SemaphoreType.DMA,
            pltpu.SemaphoreType.DMA,
        ],
    )
    def k(
        x_hbm, idx_hbm, z_hbm, out_hbm,
        src_t, dst_t, r0, r1, r2, acc,
        gs0, gs1, gs2, ss0, ss1, ss2,
    ):
        c = lax.axis_index("c")
        s = lax.axis_index("s")
        wid = s * N_CORES + c
        rows = (r0, r1, r2)
        gs = (gs0, gs1, gs2)
        ss = (ss0, ss1, ss2)

        pltpu.sync_copy(z_hbm, acc.at[pl.ds(s * ROWS_PER_TILE, ROWS_PER_TILE)])
        plsc.subcore_barrier()

        def fire_gather(j, b):
            pltpu.async_copy(x_hbm.at[src_t.at[j]], rows[b], gs[b])

        def wait_gather(j, b):
            pltpu.make_async_copy(x_hbm.at[src_t.at[j]], rows[b], gs[b]).wait()

        def fire_scatter(j, b):
            pltpu.async_copy(rows[b], acc.at[dst_t.at[j]], ss[b], add=True)

        def wait_scatter(j, b):
            pltpu.make_async_copy(rows[b], acc.at[dst_t.at[j]], ss[b]).wait()

        for seg in range(N_SEGS):
            pltpu.sync_copy(idx_hbm.at[0, wid, seg], src_t)
            pltpu.sync_copy(idx_hbm.at[1, wid, seg], dst_t)

            fire_gather(0, 0)
            fire_gather(1, 1)

            def body(t, carry):
                for kk in range(3):
                    j = 3 * t + kk
                    wait_gather(j, kk)
                    fire_scatter(j, kk)
                    prev = (kk + 2) % 3
                    if kk == 0:
                        @pl.when(t > 0)
                        def _():
                            wait_scatter(j - 1, prev)
                    else:
                        wait_scatter(j - 1, prev)
                    fire_gather(j + 2, prev)
                return carry

            lax.fori_loop(0, RING_TRIPS, body, 0)

            for j in range(SEG - 4, SEG):
                b = j % 3
                prev = (b + 2) % 3
                wait_gather(j, b)
                fire_scatter(j, b)
                wait_scatter(j - 1, prev)
                if j + 2 < SEG:
                    fire_gather(j + 2, prev)
            wait_scatter(SEG - 1, (SEG - 1) % 3)

        plsc.subcore_barrier()
        pltpu.sync_copy(
            acc.at[pl.ds(s * ROWS_PER_TILE, ROWS_PER_TILE)],
            out_hbm.at[c, pl.ds(s * ROWS_PER_TILE, ROWS_PER_TILE)],
        )

    return k(x, idx5, zrows)


def _combine(partials):
    blk = 400

    def body(p_ref, o_ref):
        o_ref[...] = p_ref[0] + p_ref[1]

    return pl.pallas_call(
        body,
        grid=(N_NODES // blk,),
        in_specs=[pl.BlockSpec((N_CORES, blk, D_FEAT), lambda i: (0, i, 0))],
        out_specs=pl.BlockSpec((blk, D_FEAT), lambda i: (i, 0)),
        out_shape=jax.ShapeDtypeStruct((N_NODES, D_FEAT), jnp.float32),
    )(partials)


def kernel(x, edge_index):
    idx5 = edge_index.reshape(2, N_WORKERS, N_SEGS, SEG, CHUNK)
    zrows = jnp.zeros((ROWS_PER_TILE, D_FEAT), jnp.float32)
    partials = _sc_partials(x, idx5, zrows)
    return _combine(partials)


# speedup vs baseline: 13.5864x; 1.0680x over previous
import functools

import jax
import jax.numpy as jnp
from jax import lax
from jax.experimental import pallas as pl
from jax.experimental.pallas import tpu as pltpu
from jax.experimental.pallas import tpu_sc as plsc

N_NODES = 10000
D_FEAT = 128
N_EDGES = 320000

N_CORES = 2
N_TILES = 16
N_WORKERS = N_CORES * N_TILES

N_PAD = 10240
CHUNK = 80
EDGES_PER_TILE = N_EDGES // N_WORKERS
N_CHUNKS = EDGES_PER_TILE // CHUNK
SEG = 25
N_SEGS = N_CHUNKS // SEG
RING_TRIPS = (SEG - 4) // 3
ROWS_PER_TILE = N_PAD // N_TILES


def _sc_partials(x, idx5):
    mesh = plsc.VectorSubcoreMesh(core_axis_name="c", subcore_axis_name="s")

    @functools.partial(
        pl.kernel,
        mesh=mesh,
        out_type=jax.ShapeDtypeStruct((N_CORES, N_PAD, D_FEAT), jnp.float32),
        scratch_types=[
            pltpu.VMEM((SEG, CHUNK), jnp.int32),
            pltpu.VMEM((SEG, CHUNK), jnp.int32),
            pltpu.VMEM((CHUNK, D_FEAT), jnp.float32),
            pltpu.VMEM((CHUNK, D_FEAT), jnp.float32),
            pltpu.VMEM((CHUNK, D_FEAT), jnp.float32),
            pltpu.VMEM_SHARED((N_PAD, D_FEAT), jnp.float32),
            pltpu.SemaphoreType.DMA,
            pltpu.SemaphoreType.DMA,
            pltpu.SemaphoreType.DMA,
            pltpu.SemaphoreType.DMA,
            pltpu.SemaphoreType.DMA,
            pltpu.SemaphoreType.DMA,
        ],
    )
    def k(
        x_hbm, idx_hbm, out_hbm,
        src_t, dst_t, r0, r1, r2, acc,
        gs0, gs1, gs2, ss0, ss1, ss2,
    ):
        c = lax.axis_index("c")
        s = lax.axis_index("s")
        wid = s * N_CORES + c
        rows = (r0, r1, r2)
        gs = (gs0, gs1, gs2)
        ss = (ss0, ss1, ss2)

        zv = jnp.zeros((16,), jnp.float32)

        def zbody(i, carry):
            for kk in range(8):
                r0[i, pl.ds(kk * 16, 16)] = zv
            return carry

        lax.fori_loop(0, CHUNK, zbody, 0)
        for kk in range(ROWS_PER_TILE // CHUNK):
            pltpu.sync_copy(
                r0, acc.at[pl.ds(s * ROWS_PER_TILE + kk * CHUNK, CHUNK)]
            )
        plsc.subcore_barrier()

        def fire_gather(j, b):
            pltpu.async_copy(x_hbm.at[src_t.at[j]], rows[b], gs[b])

        def wait_gather(j, b):
            pltpu.make_async_copy(x_hbm.at[src_t.at[j]], rows[b], gs[b]).wait()

        def fire_scatter(j, b):
            pltpu.async_copy(rows[b], acc.at[dst_t.at[j]], ss[b], add=True)

        def wait_scatter(j, b):
            pltpu.make_async_copy(rows[b], acc.at[dst_t.at[j]], ss[b]).wait()

        for seg in range(N_SEGS):
            pltpu.sync_copy(idx_hbm.at[0, wid, seg], src_t)
            pltpu.sync_copy(idx_hbm.at[1, wid, seg], dst_t)

            fire_gather(0, 0)
            fire_gather(1, 1)

            def body(t, carry):
                for kk in range(3):
                    j = 3 * t + kk
                    wait_gather(j, kk)
                    fire_scatter(j, kk)
                    prev = (kk + 2) % 3
                    if kk == 0:
                        @pl.when(t > 0)
                        def _():
                            wait_scatter(j - 1, prev)
                    else:
                        wait_scatter(j - 1, prev)
                    fire_gather(j + 2, prev)
                return carry

            lax.fori_loop(0, RING_TRIPS, body, 0)

            for j in range(SEG - 4, SEG):
                b = j % 3
                prev = (b + 2) % 3
                wait_gather(j, b)
                fire_scatter(j, b)
                wait_scatter(j - 1, prev)
                if j + 2 < SEG:
                    fire_gather(j + 2, prev)
            wait_scatter(SEG - 1, (SEG - 1) % 3)

        plsc.subcore_barrier()
        pltpu.sync_copy(
            acc.at[pl.ds(s * ROWS_PER_TILE, ROWS_PER_TILE)],
            out_hbm.at[c, pl.ds(s * ROWS_PER_TILE, ROWS_PER_TILE)],
        )

    return k(x, idx5)


COMB_ROWS = N_PAD // N_WORKERS
COMB_TAIL = N_NODES - (N_WORKERS - 1) * COMB_ROWS


def _combine(partials):
    mesh = plsc.VectorSubcoreMesh(core_axis_name="c", subcore_axis_name="s")

    @functools.partial(
        pl.kernel,
        mesh=mesh,
        out_type=jax.ShapeDtypeStruct((N_NODES, D_FEAT), jnp.float32),
        scratch_types=[
            pltpu.VMEM((COMB_ROWS, D_FEAT), jnp.float32),
            pltpu.VMEM((COMB_ROWS, D_FEAT), jnp.float32),
            pltpu.SemaphoreType.DMA,
            pltpu.SemaphoreType.DMA,
        ],
    )
    def k(p_hbm, out_hbm, a0, a1, sem0, sem1):
        c = lax.axis_index("c")
        s = lax.axis_index("s")
        wid = s * N_CORES + c
        base = wid * COMB_ROWS
        cp0 = pltpu.async_copy(p_hbm.at[0, pl.ds(base, COMB_ROWS)], a0, sem0)
        cp1 = pltpu.async_copy(p_hbm.at[1, pl.ds(base, COMB_ROWS)], a1, sem1)
        cp0.wait()
        cp1.wait()

        def body(i, carry):
            for kk in range(8):
                sl = pl.ds(kk * 16, 16)
                a0[i, sl] = a0[i, sl] + a1[i, sl]
            return carry

        lax.fori_loop(0, COMB_ROWS, body, 0)

        @pl.when(wid < N_WORKERS - 1)
        def _():
            pltpu.sync_copy(a0, out_hbm.at[pl.ds(base, COMB_ROWS)])

        @pl.when(wid == N_WORKERS - 1)
        def _():
            pltpu.sync_copy(
                a0.at[pl.ds(0, COMB_TAIL)],
                out_hbm.at[pl.ds(base, COMB_TAIL)],
            )

    return k(partials)


def kernel(x, edge_index):
    idx5 = edge_index.reshape(2, N_WORKERS, N_SEGS, SEG, CHUNK)
    partials = _sc_partials(x, idx5)
    return _combine(partials)
